# Initial kernel scaffold; baseline (speedup 1.0000x reference)
import jax
import jax.numpy as jnp
from jax.experimental import pallas as pl


def kernel(codebook_indices, penalty_matrix):
    raise NotImplementedError("write your pallas kernel here")


# speedup vs baseline: 1.2458x; 1.2458x over previous
import functools

import jax
import jax.numpy as jnp
from jax import lax
from jax.experimental import pallas as pl
from jax.experimental.pallas import tpu as pltpu
from jax.experimental.pallas import tpu_sc as plsc

_C = 8192
_B = 128
_S = 8192
_L = 16
_NW = 32
_ROWS_PER_W = _B // _NW
_VECS = _S // _L
_CHUNK = 128
_CHUNKS = _S // _CHUNK


def _sc_kernel(idx_hbm, table_hbm, out_hbm, row_v, flat_v, val_v, acc_v, sem):
    nc = 2
    wid = lax.axis_index("s") * nc + lax.axis_index("c")
    lane = lax.iota(jnp.int32, _L)

    acc_v[...] = jnp.zeros((_L,), jnp.float32)

    for r in range(_ROWS_PER_W):
        b = wid * _ROWS_PER_W + r
        pltpu.sync_copy(idx_hbm.at[b], row_v.at[pl.ds(0, _S)])
        row_v[pl.ds(_S, _L)] = jnp.zeros((_L,), jnp.int32)

        def compute_chunk(c, _):
            for k in range(_CHUNK // _L):
                off = c * _CHUNK + k * _L
                src = row_v[pl.ds(off, _L)]
                tgt = row_v[pl.ds(off + 1, _L)]
                src = jnp.clip(src, 0, _C - 1)
                tgt = jnp.clip(tgt, 0, _C - 1)
                flat_v[pl.ds(off, _L)] = src * _C + tgt
            return 0

        lax.fori_loop(0, _CHUNKS, compute_chunk, 0)

        def fire(c, _):
            pltpu.async_copy(
                table_hbm.at[flat_v.at[pl.ds(c * _CHUNK, _CHUNK)]],
                val_v.at[pl.ds(c * _CHUNK, _CHUNK)],
                sem,
            )
            return 0

        lax.fori_loop(0, _CHUNKS, fire, 0)

        def drain(c, _):
            pltpu.make_async_copy(
                table_hbm.at[flat_v.at[pl.ds(0, _CHUNK)]],
                val_v.at[pl.ds(0, _CHUNK)],
                sem,
            ).wait()
            return 0

        lax.fori_loop(0, _CHUNKS, drain, 0)

        def accum(j, _):
            v = val_v[pl.ds(j * _L, _L)]
            acc_v[...] = acc_v[...] + 1.0 / (1.0 + jnp.exp(-v))
            return 0

        lax.fori_loop(0, _VECS - 1, accum, 0)
        v = val_v[pl.ds((_VECS - 1) * _L, _L)]
        s = 1.0 / (1.0 + jnp.exp(-v))
        acc_v[...] = acc_v[...] + jnp.where(lane < _L - 1, s, 0.0)

    pltpu.sync_copy(acc_v, out_hbm.at[wid])


@jax.jit
def _run(codebook_indices, flat_table):
    mesh = plsc.VectorSubcoreMesh(core_axis_name="c", subcore_axis_name="s")
    kern = functools.partial(
        pl.kernel,
        mesh=mesh,
        out_type=jax.ShapeDtypeStruct((_NW, _L), jnp.float32),
        scratch_types=[
            pltpu.VMEM((_S + _L,), jnp.int32),
            pltpu.VMEM((_S,), jnp.int32),
            pltpu.VMEM((_S,), jnp.float32),
            pltpu.VMEM((_L,), jnp.float32),
            pltpu.SemaphoreType.DMA,
        ],
    )(_sc_kernel)
    partials = kern(codebook_indices, flat_table)
    return jnp.sum(partials) / jnp.float32(_B * (_S - 1))


def kernel(codebook_indices, penalty_matrix):
    return _run(codebook_indices, penalty_matrix.reshape(-1))
